# baseline (speedup 1.0000x reference)
import functools

import jax
import jax.numpy as jnp
from jax import lax
from jax.experimental import pallas as pl
from jax.experimental.pallas import tpu as pltpu
from jax.experimental.pallas import tpu_sc as plsc

D = 32
PACK = 8
WPT = 16
NBUF = 2
SLAB = 1 << 17


def _round_pack(lo, hi):
    u = lax.bitcast_convert_type(lo, jnp.uint32)
    u = u + 0x7FFF + ((u >> 16) & 1)
    v = lax.bitcast_convert_type(hi, jnp.uint32)
    v = v + 0x7FFF + ((v >> 16) & 1)
    word = (u >> 16) | (v & jnp.uint32(0xFFFF0000))
    return lax.bitcast_convert_type(word, jnp.float32)


@functools.lru_cache(maxsize=None)
def _dense_table_call(n_rows: int, row_blk: int):
    assert SLAB % row_blk == 0 and PACK * SLAB >= n_rows
    nblk = SLAB // row_blk
    max_blk = -(-n_rows // row_blk) - 1

    def body(*refs):
        xs = refs[:PACK]
        wlo_ref, whi_ref, blo_ref, bhi_ref, o_ref = refs[PACK:]
        lo = blo_ref[...]
        hi = bhi_ref[...]
        for j, xr in enumerate(xs):
            x = xr[...]
            lo = lo + lax.dot_general(
                x, wlo_ref[pl.ds(j * D, D), :], (((0,), (0,)), ((), ())),
                preferred_element_type=jnp.float32)
            hi = hi + lax.dot_general(
                x, whi_ref[pl.ds(j * D, D), :], (((0,), (0,)), ((), ())),
                preferred_element_type=jnp.float32)
        o_ref[...] = _round_pack(lo, hi)

    xspec = lambda j: pl.BlockSpec(
        (D, row_blk), lambda i, j=j: (0, jnp.minimum(nblk * j + i, max_blk)))
    wspec = pl.BlockSpec((PACK * D, PACK * WPT), lambda i: (0, 0))
    bspec = pl.BlockSpec((1, PACK * WPT), lambda i: (0, 0))
    return pl.pallas_call(
        body,
        grid=(nblk,),
        in_specs=[xspec(j) for j in range(PACK)] + [wspec, wspec, bspec, bspec],
        out_specs=pl.BlockSpec((row_blk, PACK * WPT), lambda i: (i, 0)),
        out_shape=jax.ShapeDtypeStruct((SLAB, PACK * WPT), jnp.float32),
    )


@functools.lru_cache(maxsize=None)
def _gather_call(n_rows: int, table_rows: int, chunk: int):
    info = plsc.get_sparse_core_info()
    nc, ns = info.num_cores, info.num_subcores
    nw = nc * ns
    rows_per_w = n_rows // nw
    n_chunks = rows_per_w // chunk
    assert n_rows % (nw * chunk) == 0 and chunk % 8 == 0
    assert n_chunks % NBUF == 0
    mesh = plsc.VectorSubcoreMesh(core_axis_name="c", subcore_axis_name="s")

    @functools.partial(
        pl.kernel,
        mesh=mesh,
        compiler_params=pltpu.CompilerParams(use_tc_tiling_on_sc=False),
        out_type=jax.ShapeDtypeStruct((n_rows, WPT), jnp.float32),
        scratch_types=[
            pltpu.VMEM((NBUF, chunk), jnp.int32),
            pltpu.VMEM((NBUF, chunk, WPT), jnp.float32),
            pltpu.SemaphoreType.DMA((NBUF,)),
        ],
    )
    def k(idx_hbm, table_hbm, out_hbm, idx_v, rows_v, gsem):
        wid = lax.axis_index("s") * nc + lax.axis_index("c")
        base = wid * rows_per_w

        def fire(j, b):
            off = base + j * chunk
            pltpu.sync_copy(idx_hbm.at[pl.ds(off, chunk)], idx_v.at[b])
            pltpu.async_copy(table_hbm.at[idx_v.at[b]], rows_v.at[b],
                             gsem.at[b])

        for b in range(NBUF):
            fire(b, b)

        def body(g, carry):
            for b in range(NBUF):
                j = g * NBUF + b
                off = base + j * chunk
                pltpu.make_async_copy(table_hbm.at[idx_v.at[b]],
                                      rows_v.at[b], gsem.at[b]).wait()
                pltpu.sync_copy(rows_v.at[b], out_hbm.at[pl.ds(off, chunk)])

                @pl.when(j + NBUF < n_chunks)
                def _():
                    fire(j + NBUF, b)

            return carry

        lax.fori_loop(0, n_chunks // NBUF, body, 0)

    return k


@functools.lru_cache(maxsize=None)
def _transpose_call(batch: int, l_len: int, b_blk: int, batch_c: int,
                    stripe: int):
    groups = l_len // PACK
    blk0 = stripe * (batch_c // b_blk)

    def body(x_ref, *rest):
        o_ref = rest[-1]
        xw = lax.bitcast_convert_type(x_ref[...], jnp.uint32)
        x3 = xw.reshape(b_blk, groups, PACK * WPT)
        for g in range(groups):
            wt = x3[:, g, :].T
            lo = lax.bitcast_convert_type(wt << 16, jnp.float32)
            hi = lax.bitcast_convert_type(wt & jnp.uint32(0xFFFF0000), jnp.float32)
            for tl in range(PACK):
                r = (g * PACK + tl) * D
                o_ref[pl.ds(r, WPT), :] = lo[tl * WPT:(tl + 1) * WPT, :]
                o_ref[pl.ds(r + WPT, WPT), :] = hi[tl * WPT:(tl + 1) * WPT, :]

    in_specs = [pl.BlockSpec((b_blk * groups, PACK * WPT), lambda i: (i, 0))]
    kwargs = {}
    if stripe:
        in_specs.append(pl.BlockSpec(memory_space=pl.ANY))
        kwargs["input_output_aliases"] = {1: 0}
    return pl.pallas_call(
        body,
        grid=(batch_c // b_blk,),
        in_specs=in_specs,
        out_specs=pl.BlockSpec((l_len * D, b_blk), lambda i: (0, blk0 + i)),
        out_shape=jax.ShapeDtypeStruct((l_len * D, batch), jnp.float32),
        **kwargs,
    )


def kernel(input_ids, table, W1, b1, W2, b2):
    B, L = input_ids.shape
    n_rows = B * L

    wc = W1.T @ W2.T
    bc = b1 @ W2.T + b2
    eye = jnp.eye(PACK, dtype=jnp.float32)
    wlo = jnp.kron(eye, wc[:, :WPT])
    whi = jnp.kron(eye, wc[:, WPT:])
    blo = jnp.tile(bc[:WPT], PACK)[None, :]
    bhi = jnp.tile(bc[WPT:], PACK)[None, :]

    t2 = _dense_table_call(table.shape[0], 8192)(
        *([table.T] * PACK), wlo, whi, blo, bhi)

    idsw = input_ids.astype(jnp.int32)
    ids2 = (((idsw & (SLAB - 1)) << 3) | (idsw >> 17)).reshape(-1)
    t2v = t2.reshape(SLAB * PACK, WPT)
    nchunk = 8
    b_c = B // nchunk
    rows_c = n_rows // nchunk
    z = None
    for c in range(nchunk):
        x_c = _gather_call(rows_c, SLAB * PACK, 1600)(
            lax.dynamic_slice_in_dim(ids2, c * rows_c, rows_c), t2v)
        x_cv = x_c.reshape(rows_c // PACK, PACK * WPT)
        if c == 0:
            z = _transpose_call(B, L, 128, b_c, 0)(x_cv)
        else:
            z = _transpose_call(B, L, 128, b_c, c)(x_cv, z)
    return (z.reshape(L, D, B).transpose(2, 0, 1),)


# speedup vs baseline: 1.0242x; 1.0242x over previous
import functools

import jax
import jax.numpy as jnp
from jax import lax
from jax.experimental import pallas as pl
from jax.experimental.pallas import tpu as pltpu
from jax.experimental.pallas import tpu_sc as plsc

D = 32
PACK = 8
WPT = 16
NBUF = 2
SLAB = 1 << 17


def _round_pack(lo, hi):
    u = lax.bitcast_convert_type(lo, jnp.uint32)
    u = u + 0x7FFF + ((u >> 16) & 1)
    v = lax.bitcast_convert_type(hi, jnp.uint32)
    v = v + 0x7FFF + ((v >> 16) & 1)
    word = (u >> 16) | (v & jnp.uint32(0xFFFF0000))
    return lax.bitcast_convert_type(word, jnp.float32)


@functools.lru_cache(maxsize=None)
def _dense_table_call(n_rows: int, row_blk: int):
    assert SLAB % row_blk == 0 and PACK * SLAB >= n_rows
    nblk = SLAB // row_blk
    max_blk = -(-n_rows // row_blk) - 1

    def body(*refs):
        xs = refs[:PACK]
        wlo_ref, whi_ref, blo_ref, bhi_ref, o_ref = refs[PACK:]
        lo = blo_ref[...]
        hi = bhi_ref[...]
        for j, xr in enumerate(xs):
            x = xr[...]
            lo = lo + lax.dot_general(
                x, wlo_ref[pl.ds(j * D, D), :], (((0,), (0,)), ((), ())),
                preferred_element_type=jnp.float32)
            hi = hi + lax.dot_general(
                x, whi_ref[pl.ds(j * D, D), :], (((0,), (0,)), ((), ())),
                preferred_element_type=jnp.float32)
        o_ref[...] = _round_pack(lo, hi)

    xspec = lambda j: pl.BlockSpec(
        (D, row_blk), lambda i, j=j: (0, jnp.minimum(nblk * j + i, max_blk)))
    wspec = pl.BlockSpec((PACK * D, PACK * WPT), lambda i: (0, 0))
    bspec = pl.BlockSpec((1, PACK * WPT), lambda i: (0, 0))
    return pl.pallas_call(
        body,
        grid=(nblk,),
        in_specs=[xspec(j) for j in range(PACK)] + [wspec, wspec, bspec, bspec],
        out_specs=pl.BlockSpec((row_blk, PACK * WPT), lambda i: (i, 0)),
        out_shape=jax.ShapeDtypeStruct((SLAB, PACK * WPT), jnp.float32),
    )


@functools.lru_cache(maxsize=None)
def _gather_call(n_rows: int, table_rows: int, chunk: int):
    info = plsc.get_sparse_core_info()
    nc, ns = info.num_cores, info.num_subcores
    nw = nc * ns
    rows_per_w = n_rows // nw
    n_chunks = rows_per_w // chunk
    assert n_rows % (nw * chunk) == 0 and chunk % 8 == 0
    assert n_chunks % NBUF == 0
    mesh = plsc.VectorSubcoreMesh(core_axis_name="c", subcore_axis_name="s")

    @functools.partial(
        pl.kernel,
        mesh=mesh,
        compiler_params=pltpu.CompilerParams(use_tc_tiling_on_sc=False),
        out_type=jax.ShapeDtypeStruct((n_rows, WPT), jnp.float32),
        scratch_types=[
            pltpu.VMEM((NBUF, chunk), jnp.int32),
            pltpu.VMEM((NBUF, chunk, WPT), jnp.float32),
            pltpu.SemaphoreType.DMA((NBUF,)),
        ],
    )
    def k(idx_hbm, table_hbm, out_hbm, idx_v, rows_v, gsem):
        wid = lax.axis_index("s") * nc + lax.axis_index("c")
        base = wid * rows_per_w

        def fire(j, b):
            off = base + j * chunk
            pltpu.sync_copy(idx_hbm.at[pl.ds(off, chunk)], idx_v.at[b])
            pltpu.async_copy(table_hbm.at[idx_v.at[b]], rows_v.at[b],
                             gsem.at[b])

        for b in range(NBUF):
            fire(b, b)

        def body(g, carry):
            for b in range(NBUF):
                j = g * NBUF + b
                off = base + j * chunk
                pltpu.make_async_copy(table_hbm.at[idx_v.at[b]],
                                      rows_v.at[b], gsem.at[b]).wait()
                pltpu.sync_copy(rows_v.at[b], out_hbm.at[pl.ds(off, chunk)])

                @pl.when(j + NBUF < n_chunks)
                def _():
                    fire(j + NBUF, b)

            return carry

        lax.fori_loop(0, n_chunks // NBUF, body, 0)

    return k


@functools.lru_cache(maxsize=None)
def _transpose_call(batch: int, l_len: int, b_blk: int, batch_c: int,
                    stripe: int):
    groups = l_len // PACK
    blk0 = stripe * (batch_c // b_blk)

    def body(x_ref, *rest):
        o_ref = rest[-1]
        xw = lax.bitcast_convert_type(x_ref[...], jnp.uint32)
        x3 = xw.reshape(b_blk, groups, PACK * WPT)
        for g in range(groups):
            wt = x3[:, g, :].T
            lo = lax.bitcast_convert_type(wt << 16, jnp.float32)
            hi = lax.bitcast_convert_type(wt & jnp.uint32(0xFFFF0000), jnp.float32)
            for tl in range(PACK):
                r = (g * PACK + tl) * D
                o_ref[pl.ds(r, WPT), :] = lo[tl * WPT:(tl + 1) * WPT, :]
                o_ref[pl.ds(r + WPT, WPT), :] = hi[tl * WPT:(tl + 1) * WPT, :]

    in_specs = [pl.BlockSpec((b_blk * groups, PACK * WPT), lambda i: (i, 0))]
    kwargs = {}
    if stripe:
        in_specs.append(pl.BlockSpec(memory_space=pl.ANY))
        kwargs["input_output_aliases"] = {1: 0}
    return pl.pallas_call(
        body,
        grid=(batch_c // b_blk,),
        in_specs=in_specs,
        out_specs=pl.BlockSpec((l_len * D, b_blk), lambda i: (0, blk0 + i)),
        out_shape=jax.ShapeDtypeStruct((l_len * D, batch), jnp.float32),
        **kwargs,
    )


def kernel(input_ids, table, W1, b1, W2, b2):
    B, L = input_ids.shape
    n_rows = B * L

    wc = W1.T @ W2.T
    bc = b1 @ W2.T + b2
    eye = jnp.eye(PACK, dtype=jnp.float32)
    wlo = jnp.kron(eye, wc[:, :WPT])
    whi = jnp.kron(eye, wc[:, WPT:])
    blo = jnp.tile(bc[:WPT], PACK)[None, :]
    bhi = jnp.tile(bc[WPT:], PACK)[None, :]

    t2 = _dense_table_call(table.shape[0], 8192)(
        *([table.T] * PACK), wlo, whi, blo, bhi)

    idsw = input_ids.astype(jnp.int32)
    ids2 = (((idsw & (SLAB - 1)) << 3) | (idsw >> 17)).reshape(-1)
    t2v = t2.reshape(SLAB * PACK, WPT)
    nchunk = 4
    b_c = B // nchunk
    rows_c = n_rows // nchunk
    z = None
    for c in range(nchunk):
        x_c = _gather_call(rows_c, SLAB * PACK, 1600)(
            lax.dynamic_slice_in_dim(ids2, c * rows_c, rows_c), t2v)
        x_cv = x_c.reshape(rows_c // PACK, PACK * WPT)
        if c == 0:
            z = _transpose_call(B, L, 128, b_c, 0)(x_cv)
        else:
            z = _transpose_call(B, L, 128, b_c, c)(x_cv, z)
    return (z.reshape(L, D, B).transpose(2, 0, 1),)


# speedup vs baseline: 1.0370x; 1.0125x over previous
import functools

import jax
import jax.numpy as jnp
from jax import lax
from jax.experimental import pallas as pl
from jax.experimental.pallas import tpu as pltpu
from jax.experimental.pallas import tpu_sc as plsc

D = 32
PACK = 8
WPT = 16
NBUF = 2
SLAB = 1 << 17


def _round_pack(lo, hi):
    u = lax.bitcast_convert_type(lo, jnp.uint32)
    u = u + 0x7FFF + ((u >> 16) & 1)
    v = lax.bitcast_convert_type(hi, jnp.uint32)
    v = v + 0x7FFF + ((v >> 16) & 1)
    word = (u >> 16) | (v & jnp.uint32(0xFFFF0000))
    return lax.bitcast_convert_type(word, jnp.float32)


@functools.lru_cache(maxsize=None)
def _dense_table_call(n_rows: int, row_blk: int):
    assert SLAB % row_blk == 0 and PACK * SLAB >= n_rows
    nblk = SLAB // row_blk
    max_blk = -(-n_rows // row_blk) - 1

    def body(*refs):
        xs = refs[:PACK]
        wlo_ref, whi_ref, blo_ref, bhi_ref, o_ref = refs[PACK:]
        lo = blo_ref[...]
        hi = bhi_ref[...]
        for j, xr in enumerate(xs):
            x = xr[...]
            lo = lo + lax.dot_general(
                x, wlo_ref[pl.ds(j * D, D), :], (((0,), (0,)), ((), ())),
                preferred_element_type=jnp.float32)
            hi = hi + lax.dot_general(
                x, whi_ref[pl.ds(j * D, D), :], (((0,), (0,)), ((), ())),
                preferred_element_type=jnp.float32)
        o_ref[...] = _round_pack(lo, hi)

    xspec = lambda j: pl.BlockSpec(
        (D, row_blk), lambda i, j=j: (0, jnp.minimum(nblk * j + i, max_blk)))
    wspec = pl.BlockSpec((PACK * D, PACK * WPT), lambda i: (0, 0))
    bspec = pl.BlockSpec((1, PACK * WPT), lambda i: (0, 0))
    return pl.pallas_call(
        body,
        grid=(nblk,),
        in_specs=[xspec(j) for j in range(PACK)] + [wspec, wspec, bspec, bspec],
        out_specs=pl.BlockSpec((row_blk, PACK * WPT), lambda i: (i, 0)),
        out_shape=jax.ShapeDtypeStruct((SLAB, PACK * WPT), jnp.float32),
    )


@functools.lru_cache(maxsize=None)
def _gather_call(n_rows: int, table_rows: int, chunk: int):
    info = plsc.get_sparse_core_info()
    nc, ns = info.num_cores, info.num_subcores
    nw = nc * ns
    rows_per_w = n_rows // nw
    n_chunks = rows_per_w // chunk
    assert n_rows % (nw * chunk) == 0 and chunk % 8 == 0
    assert n_chunks % NBUF == 0
    mesh = plsc.VectorSubcoreMesh(core_axis_name="c", subcore_axis_name="s")

    @functools.partial(
        pl.kernel,
        mesh=mesh,
        compiler_params=pltpu.CompilerParams(use_tc_tiling_on_sc=False),
        out_type=jax.ShapeDtypeStruct((n_rows, WPT), jnp.float32),
        scratch_types=[
            pltpu.VMEM((NBUF, chunk), jnp.int32),
            pltpu.VMEM((NBUF, chunk, WPT), jnp.float32),
            pltpu.SemaphoreType.DMA((NBUF,)),
        ],
    )
    def k(idx_hbm, table_hbm, out_hbm, idx_v, rows_v, gsem):
        wid = lax.axis_index("s") * nc + lax.axis_index("c")
        base = wid * rows_per_w

        def fire(j, b):
            off = base + j * chunk
            pltpu.sync_copy(idx_hbm.at[pl.ds(off, chunk)], idx_v.at[b])
            pltpu.async_copy(table_hbm.at[idx_v.at[b]], rows_v.at[b],
                             gsem.at[b])

        for b in range(NBUF):
            fire(b, b)

        def body(g, carry):
            for b in range(NBUF):
                j = g * NBUF + b
                off = base + j * chunk
                pltpu.make_async_copy(table_hbm.at[idx_v.at[b]],
                                      rows_v.at[b], gsem.at[b]).wait()
                pltpu.sync_copy(rows_v.at[b], out_hbm.at[pl.ds(off, chunk)])

                @pl.when(j + NBUF < n_chunks)
                def _():
                    fire(j + NBUF, b)

            return carry

        lax.fori_loop(0, n_chunks // NBUF, body, 0)

    return k


@functools.lru_cache(maxsize=None)
def _transpose_call(batch: int, l_len: int, b_blk: int, batch_c: int,
                    stripe: int):
    groups = l_len // PACK
    blk0 = stripe * (batch_c // b_blk)

    def body(x_ref, *rest):
        o_ref = rest[-1]
        xw = lax.bitcast_convert_type(x_ref[...], jnp.uint32)
        x3 = xw.reshape(b_blk, groups, PACK * WPT)
        for g in range(groups):
            wt = x3[:, g, :].T
            lo = lax.bitcast_convert_type(wt << 16, jnp.float32)
            hi = lax.bitcast_convert_type(wt & jnp.uint32(0xFFFF0000), jnp.float32)
            for tl in range(PACK):
                r = (g * PACK + tl) * D
                o_ref[pl.ds(r, WPT), :] = lo[tl * WPT:(tl + 1) * WPT, :]
                o_ref[pl.ds(r + WPT, WPT), :] = hi[tl * WPT:(tl + 1) * WPT, :]

    in_specs = [pl.BlockSpec((b_blk * groups, PACK * WPT), lambda i: (i, 0))]
    kwargs = {}
    if stripe:
        in_specs.append(pl.BlockSpec(memory_space=pl.ANY))
        kwargs["input_output_aliases"] = {1: 0}
    return pl.pallas_call(
        body,
        grid=(batch_c // b_blk,),
        in_specs=in_specs,
        out_specs=pl.BlockSpec((l_len * D, b_blk), lambda i: (0, blk0 + i)),
        out_shape=jax.ShapeDtypeStruct((l_len * D, batch), jnp.float32),
        **kwargs,
    )


def kernel(input_ids, table, W1, b1, W2, b2):
    B, L = input_ids.shape
    n_rows = B * L

    wc = W1.T @ W2.T
    bc = b1 @ W2.T + b2
    eye = jnp.eye(PACK, dtype=jnp.float32)
    wlo = jnp.kron(eye, wc[:, :WPT])
    whi = jnp.kron(eye, wc[:, WPT:])
    blo = jnp.tile(bc[:WPT], PACK)[None, :]
    bhi = jnp.tile(bc[WPT:], PACK)[None, :]

    t2 = _dense_table_call(table.shape[0], 8192)(
        *([table.T] * PACK), wlo, whi, blo, bhi)

    idsw = input_ids.astype(jnp.int32)
    ids2 = (((idsw & (SLAB - 1)) << 3) | (idsw >> 17)).reshape(-1)
    t2v = t2.reshape(SLAB * PACK, WPT)
    nchunk = 4
    b_c = B // nchunk
    rows_c = n_rows // nchunk
    z = None
    for c in range(nchunk):
        x_c = _gather_call(rows_c, SLAB * PACK, 1600)(
            lax.dynamic_slice_in_dim(ids2, c * rows_c, rows_c), t2v)
        x_cv = x_c.reshape(rows_c // PACK, PACK * WPT)
        if c == 0:
            z = _transpose_call(B, L, 256, b_c, 0)(x_cv)
        else:
            z = _transpose_call(B, L, 256, b_c, c)(x_cv, z)
    return (z.reshape(L, D, B).transpose(2, 0, 1),)
